# Initial kernel scaffold; baseline (speedup 1.0000x reference)
import jax
import jax.numpy as jnp
from jax.experimental import pallas as pl


def kernel(h, coords, edge_index, edge_attr, emb_in_W, emb_in_b, edge_W1, edge_b1, edge_W2, edge_b2, node_W1, node_b1, node_W2, node_b2, coord_W1, coord_b1, coord_W2, emb_out_W, emb_out_b):
    raise NotImplementedError("write your pallas kernel here")


# speedup vs baseline: 2.0459x; 2.0459x over previous
import functools

import jax
import jax.numpy as jnp
from jax import lax
from jax.experimental import pallas as pl
from jax.experimental.pallas import tpu as pltpu
from jax.experimental.pallas import tpu_sc as plsc

N = 10000
E = 320000
HID = 128
W = 144
CP = 16
NC = 2
NS = 16
NW = NC * NS
EW = E // NW
C = 80
K = EW // C
NP = 10112
TPR = NP // NS
BN = 2000
BE = 1000


def _silu(x):
    return x * jax.nn.sigmoid(x)


def _linear_body(x_ref, wt_ref, b_ref, o_ref):
    o_ref[...] = jnp.dot(x_ref[...], wt_ref[...],
                         preferred_element_type=jnp.float32) + b_ref[...]


def _linear(x, wt, b):
    return pl.pallas_call(
        _linear_body,
        grid=(N // BN,),
        in_specs=[pl.BlockSpec((BN, x.shape[1]), lambda i: (i, 0)),
                  pl.BlockSpec(wt.shape, lambda i: (0, 0)),
                  pl.BlockSpec((1, wt.shape[1]), lambda i: (0, 0))],
        out_specs=pl.BlockSpec((BN, wt.shape[1]), lambda i: (i, 0)),
        out_shape=jax.ShapeDtypeStruct((N, wt.shape[1]), jnp.float32),
    )(x, wt, b)


def _prep_body(x_ref, cp_ref, wct_ref, wrt_ref, b1_ref, p_ref, q_ref):
    x = x_ref[...]
    cp = cp_ref[...]
    pf = jnp.dot(x, wct_ref[...], preferred_element_type=jnp.float32) + b1_ref[...]
    qf = jnp.dot(x, wrt_ref[...], preferred_element_type=jnp.float32)
    p_ref[...] = jnp.concatenate([pf, -cp], axis=1)
    q_ref[...] = jnp.concatenate([qf, cp], axis=1)


def _prep(x, cpad, wct, wrt, b1):
    return pl.pallas_call(
        _prep_body,
        grid=(N // BN,),
        in_specs=[pl.BlockSpec((BN, HID), lambda i: (i, 0)),
                  pl.BlockSpec((BN, CP), lambda i: (i, 0)),
                  pl.BlockSpec((HID, HID), lambda i: (0, 0)),
                  pl.BlockSpec((HID, HID), lambda i: (0, 0)),
                  pl.BlockSpec((1, HID), lambda i: (0, 0))],
        out_specs=[pl.BlockSpec((BN, W), lambda i: (i, 0)),
                   pl.BlockSpec((BN, W), lambda i: (i, 0))],
        out_shape=[jax.ShapeDtypeStruct((N, W), jnp.float32),
                   jax.ShapeDtypeStruct((N, W), jnp.float32)],
    )(x, cpad, wct, wrt, b1)


def _edge_body(g1_ref, g2_ref, ea_ref, wd_ref, wat_ref, ew2t_ref, eb2_ref,
               cw1t_ref, cb1_ref, cw2_ref, m_ref):
    s = g1_ref[...] + g2_ref[...]
    u0 = s[:, :HID]
    cd = s[:, HID:HID + 3]
    dist = jnp.sum(cd * cd, axis=1, keepdims=True)
    pre = u0 + dist * wd_ref[...] + jnp.dot(ea_ref[...], wat_ref[...],
                                            preferred_element_type=jnp.float32)
    u = _silu(pre)
    ef = _silu(jnp.dot(u, ew2t_ref[...],
                       preferred_element_type=jnp.float32) + eb2_ref[...])
    t = _silu(jnp.dot(ef, cw1t_ref[...],
                      preferred_element_type=jnp.float32) + cb1_ref[...])
    cw = jnp.sum(t * cw2_ref[...], axis=1, keepdims=True)
    ones = jnp.ones((BE, 1), jnp.float32)
    zeros = jnp.zeros((BE, W - HID - 4), jnp.float32)
    m_ref[...] = jnp.concatenate([ef, cd * cw, ones, zeros], axis=1)


def _edge(g1, g2, ea, wd, wat, ew2t, eb2, cw1t, cb1, cw2):
    return pl.pallas_call(
        _edge_body,
        grid=(E // BE,),
        in_specs=[pl.BlockSpec((BE, W), lambda i: (i, 0)),
                  pl.BlockSpec((BE, W), lambda i: (i, 0)),
                  pl.BlockSpec((BE, 4), lambda i: (i, 0)),
                  pl.BlockSpec((1, HID), lambda i: (0, 0)),
                  pl.BlockSpec((4, HID), lambda i: (0, 0)),
                  pl.BlockSpec((HID, HID), lambda i: (0, 0)),
                  pl.BlockSpec((1, HID), lambda i: (0, 0)),
                  pl.BlockSpec((HID, HID), lambda i: (0, 0)),
                  pl.BlockSpec((1, HID), lambda i: (0, 0)),
                  pl.BlockSpec((1, HID), lambda i: (0, 0))],
        out_specs=pl.BlockSpec((BE, W), lambda i: (i, 0)),
        out_shape=jax.ShapeDtypeStruct((E, W), jnp.float32),
    )(g1, g2, ea, wd, wat, ew2t, eb2, cw1t, cb1, cw2)


def _node_body(x_ref, cp_ref, s0_ref, s1_ref, w1at_ref, w1bt_ref, b1_ref,
               w2t_ref, b2_ref, xo_ref, cpo_ref):
    x = x_ref[...]
    s = s0_ref[0] + s1_ref[0]
    agg = s[:, :HID]
    csum = s[:, HID:HID + 3]
    cnt = s[:, HID + 3:HID + 4]
    aggc = csum / jnp.maximum(cnt, 1.0)
    pre = (jnp.dot(x, w1at_ref[...], preferred_element_type=jnp.float32)
           + jnp.dot(agg, w1bt_ref[...], preferred_element_type=jnp.float32)
           + b1_ref[...])
    upd = jnp.dot(_silu(pre), w2t_ref[...],
                  preferred_element_type=jnp.float32) + b2_ref[...]
    xo_ref[...] = x + upd
    cpo_ref[...] = cp_ref[...] + jnp.concatenate(
        [aggc, jnp.zeros((BN, CP - 3), jnp.float32)], axis=1)


def _node(x, cpad, s2n, w1at, w1bt, b1, w2t, b2):
    nb = N // BN
    return pl.pallas_call(
        _node_body,
        grid=(nb,),
        in_specs=[pl.BlockSpec((BN, HID), lambda i: (i, 0)),
                  pl.BlockSpec((BN, CP), lambda i: (i, 0)),
                  pl.BlockSpec((1, BN, W), lambda i: (0, i, 0)),
                  pl.BlockSpec((1, BN, W), lambda i: (1, i, 0)),
                  pl.BlockSpec((HID, HID), lambda i: (0, 0)),
                  pl.BlockSpec((HID, HID), lambda i: (0, 0)),
                  pl.BlockSpec((1, HID), lambda i: (0, 0)),
                  pl.BlockSpec((HID, HID), lambda i: (0, 0)),
                  pl.BlockSpec((1, HID), lambda i: (0, 0))],
        out_specs=[pl.BlockSpec((BN, HID), lambda i: (i, 0)),
                   pl.BlockSpec((BN, CP), lambda i: (i, 0))],
        out_shape=[jax.ShapeDtypeStruct((N, HID), jnp.float32),
                   jax.ShapeDtypeStruct((N, CP), jnp.float32)],
    )(x, cpad, s2n, s2n, w1at, w1bt, b1, w2t, b2)


@functools.lru_cache(maxsize=1)
def _sc_mesh():
    return plsc.VectorSubcoreMesh(core_axis_name="c", subcore_axis_name="s")


def _gather_body(p_hbm, q_hbm, col_hbm, row_hbm, g1_hbm, g2_hbm,
                 idxc, idxr, buf_a, buf_b, sem_a, sem_b):
    wid = lax.axis_index("s") * NC + lax.axis_index("c")
    pltpu.sync_copy(col_hbm.at[wid], idxc)
    pltpu.sync_copy(row_hbm.at[wid], idxr)

    def body(j, carry):
        a = pltpu.async_copy(p_hbm.at[idxc.at[j]], buf_a, sem_a)
        b = pltpu.async_copy(q_hbm.at[idxr.at[j]], buf_b, sem_b)
        a.wait()
        b.wait()
        base = wid * EW + j * C
        pltpu.sync_copy(buf_a, g1_hbm.at[pl.ds(base, C)])
        pltpu.sync_copy(buf_b, g2_hbm.at[pl.ds(base, C)])
        return carry

    lax.fori_loop(0, K, body, 0)


def _sc_gather(p, q, col3, row3):
    kfn = pl.kernel(
        _gather_body,
        out_type=[jax.ShapeDtypeStruct((E, W), jnp.float32),
                  jax.ShapeDtypeStruct((E, W), jnp.float32)],
        mesh=_sc_mesh(),
        scratch_types=[pltpu.VMEM((K, C), jnp.int32),
                       pltpu.VMEM((K, C), jnp.int32),
                       pltpu.VMEM((C, W), jnp.float32),
                       pltpu.VMEM((C, W), jnp.float32),
                       pltpu.SemaphoreType.DMA,
                       pltpu.SemaphoreType.DMA],
        compiler_params=pltpu.CompilerParams(use_tc_tiling_on_sc=False),
    )
    return kfn(p, q, col3, row3)


def _scatter_body(m_hbm, col_hbm, z_hbm, s_hbm, idxc, buf, acc):
    cid = lax.axis_index("c")
    sid = lax.axis_index("s")
    wid = sid * NC + cid
    pltpu.sync_copy(z_hbm.at[pl.ds(sid * TPR, TPR)],
                    acc.at[pl.ds(sid * TPR, TPR)])
    pltpu.sync_copy(col_hbm.at[wid], idxc)
    plsc.subcore_barrier()

    def body(j, carry):
        pltpu.sync_copy(m_hbm.at[pl.ds(wid * EW + j * C, C)], buf)
        pltpu.sync_copy(buf, acc.at[idxc.at[j]], add=True)
        return carry

    lax.fori_loop(0, K, body, 0)
    plsc.subcore_barrier()
    pltpu.sync_copy(acc.at[pl.ds(sid * TPR, TPR)],
                    s_hbm.at[cid, pl.ds(sid * TPR, TPR)])


def _sc_scatter(m, col3, zeros):
    kfn = pl.kernel(
        _scatter_body,
        out_type=jax.ShapeDtypeStruct((2, NP, W), jnp.float32),
        mesh=_sc_mesh(),
        scratch_types=[pltpu.VMEM((K, C), jnp.int32),
                       pltpu.VMEM((C, W), jnp.float32),
                       pltpu.VMEM_SHARED((NP, W), jnp.float32)],
        compiler_params=pltpu.CompilerParams(use_tc_tiling_on_sc=False),
    )
    return kfn(m, col3, zeros)


def kernel(h, coords, edge_index, edge_attr, emb_in_W, emb_in_b,
           edge_W1, edge_b1, edge_W2, edge_b2,
           node_W1, node_b1, node_W2, node_b2,
           coord_W1, coord_b1, coord_W2, emb_out_W, emb_out_b):
    row3 = edge_index[0].reshape(NW, K, C)
    col3 = edge_index[1].reshape(NW, K, C)
    zeros = jnp.zeros((NP, W), jnp.float32)
    cpad = jnp.pad(coords, ((0, 0), (0, CP - 3)))

    x = _linear(h, emb_in_W.T, emb_in_b.reshape(1, HID))
    for l in range(4):
        eW1 = edge_W1[l]
        p, q = _prep(x, cpad, eW1[:, :HID].T, eW1[:, HID:2 * HID].T,
                     edge_b1[l].reshape(1, HID))
        g1, g2 = _sc_gather(p, q, col3, row3)
        m = _edge(g1, g2, edge_attr,
                  eW1[:, 2 * HID].reshape(1, HID),
                  eW1[:, 2 * HID + 1:].T,
                  edge_W2[l].T, edge_b2[l].reshape(1, HID),
                  coord_W1[l].T, coord_b1[l].reshape(1, HID),
                  coord_W2[l].reshape(1, HID))
        s2n = _sc_scatter(m, col3, zeros)
        x, cpad = _node(x, cpad, s2n,
                        node_W1[l][:, :HID].T, node_W1[l][:, HID:].T,
                        node_b1[l].reshape(1, HID),
                        node_W2[l].T, node_b2[l].reshape(1, HID))
    x = _linear(x, emb_out_W.T, emb_out_b.reshape(1, HID))
    return (x, cpad[:, :3])


# speedup vs baseline: 2.9787x; 1.4559x over previous
import functools

import jax
import jax.numpy as jnp
from jax import lax
from jax.experimental import pallas as pl
from jax.experimental.pallas import tpu as pltpu
from jax.experimental.pallas import tpu_sc as plsc

N = 10000
E = 320000
HID = 128
CP = 16
NC = 2
NS = 16
NW = NC * NS
EW = E // NW
C = 80
K = EW // C
NP = 10112
TPR = NP // NS
BN = 2000
BE = 1280


def _silu(x):
    return x * jax.nn.sigmoid(x)


def _linear_body(x_ref, wt_ref, b_ref, o_ref):
    o_ref[...] = jnp.dot(x_ref[...], wt_ref[...],
                         preferred_element_type=jnp.float32) + b_ref[...]


def _linear(x, wt, b):
    return pl.pallas_call(
        _linear_body,
        grid=(N // BN,),
        in_specs=[pl.BlockSpec((BN, x.shape[1]), lambda i: (i, 0)),
                  pl.BlockSpec(wt.shape, lambda i: (0, 0)),
                  pl.BlockSpec((1, wt.shape[1]), lambda i: (0, 0))],
        out_specs=pl.BlockSpec((BN, wt.shape[1]), lambda i: (i, 0)),
        out_shape=jax.ShapeDtypeStruct((N, wt.shape[1]), jnp.float32),
    )(x, wt, b)


def _prep_body(x_ref, wct_ref, wrt_ref, b1_ref, p_ref, q_ref):
    x = x_ref[...]
    p_ref[...] = jnp.dot(x, wct_ref[...],
                         preferred_element_type=jnp.float32) + b1_ref[...]
    q_ref[...] = jnp.dot(x, wrt_ref[...], preferred_element_type=jnp.float32)


def _prep(x, wct, wrt, b1):
    return pl.pallas_call(
        _prep_body,
        grid=(N // BN,),
        in_specs=[pl.BlockSpec((BN, HID), lambda i: (i, 0)),
                  pl.BlockSpec((HID, HID), lambda i: (0, 0)),
                  pl.BlockSpec((HID, HID), lambda i: (0, 0)),
                  pl.BlockSpec((1, HID), lambda i: (0, 0))],
        out_specs=[pl.BlockSpec((BN, HID), lambda i: (i, 0)),
                   pl.BlockSpec((BN, HID), lambda i: (i, 0))],
        out_shape=[jax.ShapeDtypeStruct((N, HID), jnp.float32),
                   jax.ShapeDtypeStruct((N, HID), jnp.float32)],
    )(x, wct, wrt, b1)


def _edge_body(g1f_ref, g2f_ref, g1c_ref, g2c_ref, ea_ref, wd_ref, wat_ref,
               ew2t_ref, eb2_ref, cw1t_ref, cb1_ref, cw2_ref, m1_ref, m2_ref):
    cd = g1c_ref[...] + g2c_ref[...]
    cd3 = cd[:, :3]
    dist = jnp.sum(cd3 * cd3, axis=1, keepdims=True)
    pre = (g1f_ref[...] + g2f_ref[...] + dist * wd_ref[...]
           + jnp.dot(ea_ref[...], wat_ref[...],
                     preferred_element_type=jnp.float32))
    u = _silu(pre)
    ef = _silu(jnp.dot(u, ew2t_ref[...],
                       preferred_element_type=jnp.float32) + eb2_ref[...])
    t = _silu(jnp.dot(ef, cw1t_ref[...],
                      preferred_element_type=jnp.float32) + cb1_ref[...])
    cw = jnp.sum(t * cw2_ref[...], axis=1, keepdims=True)
    ones = jnp.ones((BE, 1), jnp.float32)
    zeros = jnp.zeros((BE, CP - 4), jnp.float32)
    m1_ref[...] = ef
    m2_ref[...] = jnp.concatenate([cd3 * cw, ones, zeros], axis=1)


def _edge(g1f, g2f, g1c, g2c, ea, wd, wat, ew2t, eb2, cw1t, cb1, cw2):
    return pl.pallas_call(
        _edge_body,
        grid=(E // BE,),
        in_specs=[pl.BlockSpec((BE, HID), lambda i: (i, 0)),
                  pl.BlockSpec((BE, HID), lambda i: (i, 0)),
                  pl.BlockSpec((BE, CP), lambda i: (i, 0)),
                  pl.BlockSpec((BE, CP), lambda i: (i, 0)),
                  pl.BlockSpec((BE, 4), lambda i: (i, 0)),
                  pl.BlockSpec((1, HID), lambda i: (0, 0)),
                  pl.BlockSpec((4, HID), lambda i: (0, 0)),
                  pl.BlockSpec((HID, HID), lambda i: (0, 0)),
                  pl.BlockSpec((1, HID), lambda i: (0, 0)),
                  pl.BlockSpec((HID, HID), lambda i: (0, 0)),
                  pl.BlockSpec((1, HID), lambda i: (0, 0)),
                  pl.BlockSpec((1, HID), lambda i: (0, 0))],
        out_specs=[pl.BlockSpec((BE, HID), lambda i: (i, 0)),
                   pl.BlockSpec((BE, CP), lambda i: (i, 0))],
        out_shape=[jax.ShapeDtypeStruct((E, HID), jnp.float32),
                   jax.ShapeDtypeStruct((E, CP), jnp.float32)],
    )(g1f, g2f, g1c, g2c, ea, wd, wat, ew2t, eb2, cw1t, cb1, cw2)


def _node_body(x_ref, cp_ref, s1a_ref, s1b_ref, s2a_ref, s2b_ref,
               w1at_ref, w1bt_ref, b1_ref, w2t_ref, b2_ref,
               xo_ref, cpo_ref, cno_ref):
    x = x_ref[...]
    agg = s1a_ref[0] + s1b_ref[0]
    t2 = s2a_ref[0] + s2b_ref[0]
    csum = t2[:, :3]
    cnt = t2[:, 3:4]
    aggc = csum / jnp.maximum(cnt, 1.0)
    pre = (jnp.dot(x, w1at_ref[...], preferred_element_type=jnp.float32)
           + jnp.dot(agg, w1bt_ref[...], preferred_element_type=jnp.float32)
           + b1_ref[...])
    upd = jnp.dot(_silu(pre), w2t_ref[...],
                  preferred_element_type=jnp.float32) + b2_ref[...]
    xo_ref[...] = x + upd
    cpo = cp_ref[...] + jnp.concatenate(
        [aggc, jnp.zeros((BN, CP - 3), jnp.float32)], axis=1)
    cpo_ref[...] = cpo
    cno_ref[...] = -cpo


def _node(x, cpad, s1, s2, w1at, w1bt, b1, w2t, b2):
    return pl.pallas_call(
        _node_body,
        grid=(N // BN,),
        in_specs=[pl.BlockSpec((BN, HID), lambda i: (i, 0)),
                  pl.BlockSpec((BN, CP), lambda i: (i, 0)),
                  pl.BlockSpec((1, BN, HID), lambda i: (0, i, 0)),
                  pl.BlockSpec((1, BN, HID), lambda i: (1, i, 0)),
                  pl.BlockSpec((1, BN, CP), lambda i: (0, i, 0)),
                  pl.BlockSpec((1, BN, CP), lambda i: (1, i, 0)),
                  pl.BlockSpec((HID, HID), lambda i: (0, 0)),
                  pl.BlockSpec((HID, HID), lambda i: (0, 0)),
                  pl.BlockSpec((1, HID), lambda i: (0, 0)),
                  pl.BlockSpec((HID, HID), lambda i: (0, 0)),
                  pl.BlockSpec((1, HID), lambda i: (0, 0))],
        out_specs=[pl.BlockSpec((BN, HID), lambda i: (i, 0)),
                   pl.BlockSpec((BN, CP), lambda i: (i, 0)),
                   pl.BlockSpec((BN, CP), lambda i: (i, 0))],
        out_shape=[jax.ShapeDtypeStruct((N, HID), jnp.float32),
                   jax.ShapeDtypeStruct((N, CP), jnp.float32),
                   jax.ShapeDtypeStruct((N, CP), jnp.float32)],
    )(x, cpad, s1, s1, s2, s2, w1at, w1bt, b1, w2t, b2)


@functools.lru_cache(maxsize=1)
def _sc_mesh():
    return plsc.VectorSubcoreMesh(core_axis_name="c", subcore_axis_name="s")


def _gather_body(pf_hbm, qf_hbm, pc_hbm, qc_hbm, col_hbm, row_hbm,
                 g1f_hbm, g2f_hbm, g1c_hbm, g2c_hbm,
                 idxc, idxr, buf_af, buf_bf, buf_ac, buf_bc,
                 sem_a, sem_b, sem_c, sem_d):
    wid = lax.axis_index("s") * NC + lax.axis_index("c")
    pltpu.sync_copy(col_hbm.at[wid], idxc)
    pltpu.sync_copy(row_hbm.at[wid], idxr)

    def body(j, carry):
        a = pltpu.async_copy(pf_hbm.at[idxc.at[j]], buf_af, sem_a)
        b = pltpu.async_copy(qf_hbm.at[idxr.at[j]], buf_bf, sem_b)
        c = pltpu.async_copy(pc_hbm.at[idxc.at[j]], buf_ac, sem_c)
        d = pltpu.async_copy(qc_hbm.at[idxr.at[j]], buf_bc, sem_d)
        a.wait()
        b.wait()
        c.wait()
        d.wait()
        base = wid * EW + j * C
        pltpu.sync_copy(buf_af, g1f_hbm.at[pl.ds(base, C)])
        pltpu.sync_copy(buf_bf, g2f_hbm.at[pl.ds(base, C)])
        pltpu.sync_copy(buf_ac, g1c_hbm.at[pl.ds(base, C)])
        pltpu.sync_copy(buf_bc, g2c_hbm.at[pl.ds(base, C)])
        return carry

    lax.fori_loop(0, K, body, 0)


def _sc_gather(pf, qf, pc, qc, col3, row3):
    kfn = pl.kernel(
        _gather_body,
        out_type=[jax.ShapeDtypeStruct((E, HID), jnp.float32),
                  jax.ShapeDtypeStruct((E, HID), jnp.float32),
                  jax.ShapeDtypeStruct((E, CP), jnp.float32),
                  jax.ShapeDtypeStruct((E, CP), jnp.float32)],
        mesh=_sc_mesh(),
        scratch_types=[pltpu.VMEM((K, C), jnp.int32),
                       pltpu.VMEM((K, C), jnp.int32),
                       pltpu.VMEM((C, HID), jnp.float32),
                       pltpu.VMEM((C, HID), jnp.float32),
                       pltpu.VMEM((C, CP), jnp.float32),
                       pltpu.VMEM((C, CP), jnp.float32),
                       pltpu.SemaphoreType.DMA,
                       pltpu.SemaphoreType.DMA,
                       pltpu.SemaphoreType.DMA,
                       pltpu.SemaphoreType.DMA],
        compiler_params=pltpu.CompilerParams(use_tc_tiling_on_sc=False),
    )
    return kfn(pf, qf, pc, qc, col3, row3)


def _scatter_body(m1_hbm, m2_hbm, col_hbm, z1_hbm, z2_hbm, s1_hbm, s2_hbm,
                  idxc, buf1, buf2, acc1, acc2):
    cid = lax.axis_index("c")
    sid = lax.axis_index("s")
    wid = sid * NC + cid
    pltpu.sync_copy(z1_hbm.at[pl.ds(sid * TPR, TPR)],
                    acc1.at[pl.ds(sid * TPR, TPR)])
    pltpu.sync_copy(z2_hbm.at[pl.ds(sid * TPR, TPR)],
                    acc2.at[pl.ds(sid * TPR, TPR)])
    pltpu.sync_copy(col_hbm.at[wid], idxc)
    plsc.subcore_barrier()

    def body(j, carry):
        base = wid * EW + j * C
        pltpu.sync_copy(m1_hbm.at[pl.ds(base, C)], buf1)
        pltpu.sync_copy(m2_hbm.at[pl.ds(base, C)], buf2)
        pltpu.sync_copy(buf1, acc1.at[idxc.at[j]], add=True)
        pltpu.sync_copy(buf2, acc2.at[idxc.at[j]], add=True)
        return carry

    lax.fori_loop(0, K, body, 0)
    plsc.subcore_barrier()
    pltpu.sync_copy(acc1.at[pl.ds(sid * TPR, TPR)],
                    s1_hbm.at[cid, pl.ds(sid * TPR, TPR)])
    pltpu.sync_copy(acc2.at[pl.ds(sid * TPR, TPR)],
                    s2_hbm.at[cid, pl.ds(sid * TPR, TPR)])


def _sc_scatter(m1, m2, col3, zeros1, zeros2):
    kfn = pl.kernel(
        _scatter_body,
        out_type=[jax.ShapeDtypeStruct((2, NP, HID), jnp.float32),
                  jax.ShapeDtypeStruct((2, NP, CP), jnp.float32)],
        mesh=_sc_mesh(),
        scratch_types=[pltpu.VMEM((K, C), jnp.int32),
                       pltpu.VMEM((C, HID), jnp.float32),
                       pltpu.VMEM((C, CP), jnp.float32),
                       pltpu.VMEM_SHARED((NP, HID), jnp.float32),
                       pltpu.VMEM_SHARED((NP, CP), jnp.float32)],
        compiler_params=pltpu.CompilerParams(use_tc_tiling_on_sc=False),
    )
    return kfn(m1, m2, col3, zeros1, zeros2)


def kernel(h, coords, edge_index, edge_attr, emb_in_W, emb_in_b,
           edge_W1, edge_b1, edge_W2, edge_b2,
           node_W1, node_b1, node_W2, node_b2,
           coord_W1, coord_b1, coord_W2, emb_out_W, emb_out_b):
    row3 = edge_index[0].reshape(NW, K, C)
    col3 = edge_index[1].reshape(NW, K, C)
    zeros1 = jnp.zeros((NP, HID), jnp.float32)
    zeros2 = jnp.zeros((NP, CP), jnp.float32)
    cpad = jnp.pad(coords, ((0, 0), (0, CP - 3)))
    cneg = -cpad

    x = _linear(h, emb_in_W.T, emb_in_b.reshape(1, HID))
    for l in range(4):
        eW1 = edge_W1[l]
        pf, qf = _prep(x, eW1[:, :HID].T, eW1[:, HID:2 * HID].T,
                       edge_b1[l].reshape(1, HID))
        g1f, g2f, g1c, g2c = _sc_gather(pf, qf, cneg, cpad, col3, row3)
        m1, m2 = _edge(g1f, g2f, g1c, g2c, edge_attr,
                       eW1[:, 2 * HID].reshape(1, HID),
                       eW1[:, 2 * HID + 1:].T,
                       edge_W2[l].T, edge_b2[l].reshape(1, HID),
                       coord_W1[l].T, coord_b1[l].reshape(1, HID),
                       coord_W2[l].reshape(1, HID))
        s1, s2 = _sc_scatter(m1, m2, col3, zeros1, zeros2)
        x, cpad, cneg = _node(x, cpad, s1, s2,
                              node_W1[l][:, :HID].T, node_W1[l][:, HID:].T,
                              node_b1[l].reshape(1, HID),
                              node_W2[l].T, node_b2[l].reshape(1, HID))
    x = _linear(x, emb_out_W.T, emb_out_b.reshape(1, HID))
    return (x, cpad[:, :3])
